# baseline (speedup 1.0000x reference)
import functools

import jax
import jax.numpy as jnp
from jax import lax
from jax.experimental import pallas as pl
from jax.experimental.pallas import tpu as pltpu
from jax.experimental.pallas import tpu_sc as plsc

N_FIELDS = 26
VOCAB = 100000
EMBED_DIM = 32
BATCH = 16384

NUM_WORKERS = 32
B_PER_W = BATCH // NUM_WORKERS
LANES = 16
NVREG = B_PER_W // LANES
N_CHUNKS = 10
CHUNK_R = VOCAB // N_CHUNKS
SLICE = 632
PAD = 64
BINBUF = 1152


def _body(inputs_hbm, tables_hbm, out_hbm,
          idx_all, bidx_v, cnt_v, inv_v, rows_b, rows_o, spmem,
          bstart_v, bcnt_v, sems_s, sems_g, sems_w):
    wid = lax.axis_index("s") * 2 + lax.axis_index("c")
    base = wid * B_PER_W
    lanes = lax.iota(jnp.int32, LANES)
    ones = jnp.ones((LANES,), jnp.int32)
    zeros16 = jnp.zeros((LANES,), jnp.int32)
    sstart = jnp.minimum(lax.axis_index("s") * SLICE, CHUNK_R - SLICE)

    pltpu.sync_copy(inputs_hbm.at[:, wid], idx_all)

    def fire_stream(f, q, buf):
        src = f * VOCAB + q * CHUNK_R + sstart
        return pltpu.async_copy(
            tables_hbm.at[pl.ds(src, SLICE)],
            spmem.at[buf, pl.ds(sstart, SLICE)],
            sems_s[buf])

    def wait_stream(buf):
        pltpu.make_async_copy(
            tables_hbm.at[pl.ds(0, SLICE)],
            spmem.at[buf, pl.ds(0, SLICE)],
            sems_s[buf]).wait()

    def wait_write(buf):
        pltpu.make_async_copy(
            rows_o.at[buf],
            out_hbm.at[pl.ds(0, B_PER_W), pl.ds(0, EMBED_DIM)],
            sems_w[buf]).wait()

    fire_stream(0, 0, 0)
    fire_stream(0, 1, 1)

    def field_body(f, _):
        def zero_cnt(b, _):
            cnt_v[pl.ds(b * LANES, LANES)] = zeros16
            return 0
        lax.fori_loop(0, N_CHUNKS, zero_cnt, 0)

        def zero_bidx(c, _):
            bidx_v[pl.ds(c * LANES, LANES)] = zeros16
            return 0
        lax.fori_loop(0, (BINBUF + PAD) // LANES, zero_bidx, 0)

        def bin_of(v):
            b = (v >= CHUNK_R).astype(jnp.int32)
            for t in range(2, N_CHUNKS):
                b = b + (v >= t * CHUNK_R).astype(jnp.int32)
            return b

        def p1(c, _):
            v = idx_all[f, pl.ds(c * LANES, LANES)]
            plsc.addupdate_scatter(cnt_v, [bin_of(v) * LANES + lanes], ones)
            return 0
        lax.fori_loop(0, NVREG, p1, 0)

        def p2(b, s):
            c16 = cnt_v[pl.ds(b * LANES, LANES)]
            ex = plsc.cumsum(c16) - c16
            cnt_v[pl.ds(b * LANES, LANES)] = ex + s
            tot = jnp.sum(c16)
            bstart_v[b] = s
            bcnt_v[b] = tot
            return lax.bitwise_and(s + tot + PAD - 1, jnp.int32(-PAD))
        lax.fori_loop(0, N_CHUNKS, p2, jnp.int32(0))

        def p3(c, _):
            v = idx_all[f, pl.ds(c * LANES, LANES)]
            b = bin_of(v)
            slot = b * LANES + lanes
            o = plsc.load_gather(cnt_v, [slot])
            plsc.store_scatter(cnt_v, [slot], o + 1)
            plsc.store_scatter(bidx_v, [o], v - b * CHUNK_R)
            plsc.store_scatter(inv_v, [c * LANES + lanes], o)
            return 0
        lax.fori_loop(0, NVREG, p3, 0)

        for q in range(N_CHUNKS):
            buf = q % 2
            wait_stream(buf)
            plsc.subcore_barrier()

            s0 = bstart_v[q]
            nq = lax.shift_right_logical(bcnt_v[q] + PAD - 1, 6)

            def g_body(k, _, buf=buf, s0=s0):
                off = pl.multiple_of(s0 + k * PAD, PAD)
                pltpu.async_copy(
                    spmem.at[buf].at[bidx_v.at[pl.ds(off, PAD)]],
                    rows_b.at[pl.ds(off, PAD)],
                    sems_g).wait()
                return 0
            lax.fori_loop(0, nq, g_body, 0)

            plsc.subcore_barrier()
            if q < N_CHUNKS - 2:
                fire_stream(f, q + 2, buf)
            else:

                @pl.when(f < N_FIELDS - 1)
                def _():
                    fire_stream(f + 1, q - (N_CHUNKS - 2), buf)

        fpar = lax.bitwise_and(f, 1)

        def emit_out(ob):
            @pl.when(f >= 2)
            def _():
                wait_write(ob)

            def r_body(c, _):
                pvec = inv_v[pl.ds(c * LANES, LANES)]
                rvec = c * LANES + lanes
                for d in range(EMBED_DIM):
                    dcol = jnp.full((LANES,), d, jnp.int32)
                    val = plsc.load_gather(rows_b, [pvec, dcol])
                    plsc.store_scatter(rows_o.at[ob], [rvec, dcol], val)
                return 0
            lax.fori_loop(0, NVREG, r_body, 0)

            pltpu.async_copy(
                rows_o.at[ob],
                out_hbm.at[pl.ds(base, B_PER_W),
                           pl.ds(f * EMBED_DIM, EMBED_DIM)],
                sems_w[ob])

        @pl.when(fpar == 0)
        def _():
            emit_out(0)

        @pl.when(fpar == 1)
        def _():
            emit_out(1)
        return 0

    lax.fori_loop(0, N_FIELDS, field_body, 0)
    wait_write(0)
    wait_write(1)


@jax.jit
def _lookup(inputs3, tables_flat):
    mesh = plsc.VectorSubcoreMesh(core_axis_name="c", subcore_axis_name="s")
    f = functools.partial(
        pl.kernel,
        mesh=mesh,
        out_type=jax.ShapeDtypeStruct((BATCH, N_FIELDS * EMBED_DIM),
                                      jnp.float32),
        scratch_types=[
            pltpu.VMEM((N_FIELDS, B_PER_W), jnp.int32),
            pltpu.VMEM((BINBUF + PAD,), jnp.int32),
            pltpu.VMEM((N_CHUNKS * LANES,), jnp.int32),
            pltpu.VMEM((B_PER_W,), jnp.int32),
            pltpu.VMEM((BINBUF + PAD, EMBED_DIM), jnp.float32),
            pltpu.VMEM((2, B_PER_W, EMBED_DIM), jnp.float32),
            pltpu.VMEM_SHARED((2, CHUNK_R, EMBED_DIM), jnp.float32),
            pltpu.SMEM((8,), jnp.int32),
            pltpu.SMEM((8,), jnp.int32),
            [pltpu.SemaphoreType.DMA] * 2,
            pltpu.SemaphoreType.DMA,
            [pltpu.SemaphoreType.DMA] * 2,
        ],
        compiler_params=pltpu.CompilerParams(
            use_tc_tiling_on_sc=False, needs_layout_passes=False),
    )(_body)
    return f(inputs3, tables_flat)


def kernel(inputs, tables):
    inputs3 = inputs.astype(jnp.int32).reshape(N_FIELDS, NUM_WORKERS, B_PER_W)
    tables_flat = tables.reshape(N_FIELDS * VOCAB, EMBED_DIM)
    return _lookup(inputs3, tables_flat)


# speedup vs baseline: 1.7016x; 1.7016x over previous
import functools

import jax
import jax.numpy as jnp
from jax import lax
from jax.experimental import pallas as pl
from jax.experimental.pallas import tpu as pltpu
from jax.experimental.pallas import tpu_sc as plsc

N_FIELDS = 26
VOCAB = 100000
EMBED_DIM = 32
BATCH = 16384

NUM_WORKERS = 32
B_PER_W = BATCH // NUM_WORKERS
CHUNK = 128
N_CHUNKS = B_PER_W // CHUNK
LANES = 16
NBUF = 6


def _body(inputs_hbm, tables_hbm, out_hbm, idx_all, rows, sems_g, sems_w):
    wid = lax.axis_index("s") * 2 + lax.axis_index("c")
    base = wid * B_PER_W

    pltpu.sync_copy(inputs_hbm.at[:, wid], idx_all)

    for i in range(N_FIELDS):
        off = jnp.full((LANES,), i * VOCAB, dtype=jnp.int32)
        for c in range(B_PER_W // LANES):
            sl = pl.ds(c * LANES, LANES)
            idx_all[i, sl] = idx_all[i, sl] + off

    def fire_gather(i):
        b = i % NBUF
        return [
            pltpu.async_copy(
                tables_hbm.at[idx_all.at[i]],
                rows.at[b],
                sems_g[b],
            )
        ]

    def fire_write(i):
        b = i % NBUF
        return pltpu.async_copy(
            rows.at[b],
            out_hbm.at[pl.ds(base, B_PER_W), pl.ds(i * EMBED_DIM, EMBED_DIM)],
            sems_w[b],
        )

    gathers = {i: fire_gather(i) for i in range(NBUF - 1)}
    writes = {}
    for j in range(N_FIELDS):
        nxt = j + NBUF - 1
        if nxt < N_FIELDS:
            if j >= 1:
                writes.pop(j - 1).wait()
            gathers[nxt] = fire_gather(nxt)
        for h in gathers.pop(j):
            h.wait()
        writes[j] = fire_write(j)
    for j in sorted(writes):
        writes.pop(j).wait()


@jax.jit
def _lookup(inputs3, tables_flat):
    mesh = plsc.VectorSubcoreMesh(core_axis_name="c", subcore_axis_name="s")
    f = functools.partial(
        pl.kernel,
        mesh=mesh,
        out_type=jax.ShapeDtypeStruct((BATCH, N_FIELDS * EMBED_DIM), jnp.float32),
        scratch_types=[
            pltpu.VMEM((N_FIELDS, B_PER_W), jnp.int32),
            pltpu.VMEM((NBUF, B_PER_W, EMBED_DIM), jnp.float32),
            [pltpu.SemaphoreType.DMA] * NBUF,
            [pltpu.SemaphoreType.DMA] * NBUF,
        ],
        compiler_params=pltpu.CompilerParams(use_tc_tiling_on_sc=False),
    )(_body)
    return f(inputs3, tables_flat)


def kernel(inputs, tables):
    inputs3 = inputs.astype(jnp.int32).reshape(N_FIELDS, NUM_WORKERS, B_PER_W)
    tables_flat = tables.reshape(N_FIELDS * VOCAB, EMBED_DIM)
    return _lookup(inputs3, tables_flat)
